# Initial kernel scaffold; baseline (speedup 1.0000x reference)
import jax
import jax.numpy as jnp
from jax.experimental import pallas as pl


def kernel(hx, current_location, y_path, image_data, W_cnn, b_cnn, W_vel, b_vel, W_scf, b_scf, W_ih, W_hh, b_ih, b_hh, W_dy, b_dy, W_score, b_score):
    raise NotImplementedError("write your pallas kernel here")


# speedup vs baseline: 2.5575x; 2.5575x over previous
import functools
import math

import jax
import jax.numpy as jnp
from jax import lax
from jax.experimental import pallas as pl
from jax.experimental.pallas import tpu as pltpu
from jax.experimental.pallas import tpu_sc as plsc

_K = 4
_BATCH = 16
_NA = 4
_BN = _BATCH * _NA
_R = _K * _BN
_HZ = 10.0
_SEQ = 40
_SPR = 6
_SPT = 6
_NB = _SPR * _SPT
_RMIN = 0.5
_RMAX = 4.0
_RSTEP = (_RMAX - _RMIN) / _SPR
_HID = 48
_HH = 80
_NPIX = _BATCH * _HH * _HH
_NGAT = _SEQ * _R
_F32 = jnp.float32
_PREC = lax.Precision.HIGHEST


def _conv_body(x_ref, w_ref, b_ref, o_ref):
    acc = jnp.dot(x_ref[...], w_ref[...], preferred_element_type=_F32,
                  precision=_PREC)
    relu = jnp.maximum(acc + b_ref[...], 0.0)
    o_ref[...] = jnp.concatenate(
        [relu, jnp.zeros((relu.shape[0], 96), _F32)], axis=1)


def _conv_feature_table(x9, w9, b2, interpret=False):
    blk = _NPIX // 8
    return pl.pallas_call(
        _conv_body,
        grid=(8,),
        in_specs=[
            pl.BlockSpec((blk, 36), lambda i: (i, 0)),
            pl.BlockSpec((36, 32), lambda i: (0, 0)),
            pl.BlockSpec((1, 32), lambda i: (0, 0)),
        ],
        out_specs=pl.BlockSpec((blk, 128), lambda i: (i, 0)),
        out_shape=jax.ShapeDtypeStruct((_NPIX, 128), _F32),
        interpret=interpret,
    )(x9, w9, b2)


def _geom_body(pxt_ref, pyt_ref, pxf_ref, pyf_ref, pvxf_ref, pvyf_ref,
               wv0_ref, wv1_ref, bv_ref,
               oh1_ref, oh2_ref, oh3_ref, yfv_ref, pix_ref):
    px = pxt_ref[...]
    py = pyt_ref[...]

    oh_refs = (oh1_ref, oh2_ref, oh3_ref)
    raws = []
    bins_iota = lax.broadcasted_iota(jnp.int32, (_NA, _NB, px.shape[1]), 1)
    for s in (1, 2, 3):
        tx = jnp.concatenate([px[s:], px[:s]], axis=0)
        ty = jnp.concatenate([py[s:], py[:s]], axis=0)
        cx = tx - px
        cy = ty - py
        dist = jnp.sqrt(cx * cx + cy * cy)
        mf = jnp.where((dist <= _RMAX) & (dist >= _RMIN), 1.0, 0.0)
        dd = jnp.where(dist < 1e-10, 1e-10, dist)
        a = jnp.clip(cx / dd, -1.0, 1.0)
        up = (a <= 0.5).astype(jnp.int32) + (a <= -0.5).astype(jnp.int32) \
            + (a <= -1.0).astype(jnp.int32)
        lo = 3 + (a >= -0.5).astype(jnp.int32) + (a >= 0.5).astype(jnp.int32) \
            + (a >= 1.0).astype(jnp.int32)
        vb = jnp.clip(jnp.where(cy < -0.01, lo, up), 0, _SPT - 1)
        ub = jnp.clip(((dist - _RMIN) / _RSTEP).astype(jnp.int32), 0, _SPR - 1)
        li = ub * _SPT + vb
        raw = jnp.where(li[:, None, :] == bins_iota, 1.0, 0.0) \
            * mf[:, None, :]
        raws.append(raw)

    cnt = raws[0] + raws[1] + raws[2]
    den = jnp.where(cnt == 0.0, 1.0, cnt)
    for s in range(3):
        oh_refs[s][...] = raws[s] / den

    vx = (pxf_ref[...] - pvxf_ref[...]) * _HZ
    vy = (pyf_ref[...] - pvyf_ref[...]) * _HZ
    yfv_ref[...] = (vx * wv0_ref[...] + vy * wv1_ref[...] + bv_ref[...])

    pxf = pxf_ref[...]
    pyf = pyf_ref[...]
    u = jnp.clip(_HH // 2 - pyf.astype(jnp.int32), 0, _HH - 1)
    v = jnp.clip(pxf.astype(jnp.int32), 0, _HH - 1)
    scene = (lax.broadcasted_iota(jnp.int32, pxf.shape, 1) // _NA) % _BATCH
    pix_ref[...] = scene * (_HH * _HH) + u * _HH + v


def _geometry(pxt, pyt, pxf, pyf, pvxf, pvyf, wv0, wv1, bv, interpret=False):
    n = pxt.shape[1]
    m = pxf.shape[1]
    return pl.pallas_call(
        _geom_body,
        out_shape=(
            jax.ShapeDtypeStruct((_NA, _NB, n), _F32),
            jax.ShapeDtypeStruct((_NA, _NB, n), _F32),
            jax.ShapeDtypeStruct((_NA, _NB, n), _F32),
            jax.ShapeDtypeStruct((16, m), _F32),
            jax.ShapeDtypeStruct((1, m), jnp.int32),
        ),
        interpret=interpret,
    )(pxt, pyt, pxf, pyf, pvxf, pvyf, wv0, wv1, bv)


def _sc_gather(table, idx):
    nw = 32
    bpw = _NGAT // nw
    mesh = plsc.VectorSubcoreMesh(core_axis_name="c", subcore_axis_name="s",
                                  num_cores=2, num_subcores=16)

    @functools.partial(
        pl.kernel, mesh=mesh,
        out_type=jax.ShapeDtypeStruct((_NGAT, 128), _F32),
        scratch_types=[
            pltpu.VMEM((bpw,), jnp.int32),
            pltpu.VMEM((bpw, 128), _F32),
            pltpu.SemaphoreType.DMA,
        ],
    )
    def gather_k(table_hbm, idx_hbm, out_hbm, idx_v, rows_v, sem):
        wid = lax.axis_index("s") * 2 + lax.axis_index("c")
        base = wid * bpw
        pltpu.sync_copy(idx_hbm.at[pl.ds(base, bpw)], idx_v)
        pltpu.async_copy(table_hbm.at[idx_v], rows_v, sem).wait()
        pltpu.sync_copy(rows_v, out_hbm.at[pl.ds(base, bpw)])

    return gather_k(table, idx)


def _gru_body(lhalf_ref, ohs_ref, w2_ref, e36_ref, s36_ref,
              wih_ref, whh_ref, bih_ref, bhh_ref,
              wdy_ref, bdy_ref, wsc_ref, bsc_ref, hx0_ref,
              dy_ref, sc_ref, hx_s, hs_s):
    it = pl.program_id(0)

    @pl.when(it == 0)
    def _init():
        hx_s[...] = hx0_ref[...]
        hs_s[...] = jnp.zeros_like(hs_s)

    hx = hx_s[...]
    hid0 = jnp.concatenate([hx[j * _BN: j * _BN + _BATCH] for j in range(_NA)],
                           axis=0)

    q0 = jnp.dot(hid0, w2_ref[...], preferred_element_type=_F32,
                 precision=_PREC)

    acc = jnp.zeros((_R, 36 * _HID), dtype=_F32)
    for s in (1, 2, 3):
        qs = jnp.concatenate([q0[s * _BATCH:], q0[:s * _BATCH]], axis=0)
        qb = jnp.broadcast_to(
            qs.reshape(_NA, 1, _BATCH, 36 * _HID),
            (_NA, _K, _BATCH, 36 * _HID)).reshape(_R, 36 * _HID)
        ohc = ohs_ref[0, :, s - 1, :]
        ohx = jnp.dot(ohc, e36_ref[...], preferred_element_type=_F32,
                      precision=_PREC)
        acc = acc + ohx * qb
    rhalf = jnp.dot(acc, s36_ref[...], preferred_element_type=_F32,
                    precision=_PREC)

    x_i = jnp.concatenate([lhalf_ref[0], rhalf], axis=1)
    gi = jnp.dot(x_i, wih_ref[...], preferred_element_type=_F32,
                 precision=_PREC) + bih_ref[...]
    gh = jnp.dot(hx, whh_ref[...], preferred_element_type=_F32,
                 precision=_PREC) + bhh_ref[...]
    r = jax.nn.sigmoid(gi[:, :_HID] + gh[:, :_HID])
    z = jax.nn.sigmoid(gi[:, _HID:2 * _HID] + gh[:, _HID:2 * _HID])
    n = jnp.tanh(gi[:, 2 * _HID:] + r * gh[:, 2 * _HID:])
    hxn = (1.0 - z) * n + z * hx
    hsn = hs_s[...] + hxn
    hx_s[...] = hxn
    hs_s[...] = hsn

    dy_ref[...] = jnp.dot(hxn, wdy_ref[...], preferred_element_type=_F32,
                          precision=_PREC) + bdy_ref[...]
    sc_ref[...] = jnp.dot(hsn, wsc_ref[...], preferred_element_type=_F32,
                          precision=_PREC) + _SEQ * bsc_ref[...]


def _recurrence(lhalf_all, ohs_all, w2, e36, s36, wih, whh, bih, bhh,
                wdy, bdy, wsc, bsc, hx0, interpret=False):
    full = lambda shape: pl.BlockSpec(shape, lambda i: tuple(0 for _ in shape))
    return pl.pallas_call(
        _gru_body,
        grid=(_SEQ,),
        in_specs=[
            pl.BlockSpec((1, _R, _HID), lambda i: (i, 0, 0)),
            pl.BlockSpec((1, _R, 3, _NB), lambda i: (i, 0, 0, 0)),
            full((_HID, 36 * _HID)),
            full((_NB, 36 * _HID)),
            full((36 * _HID, _HID)),
            full((2 * _HID, 3 * _HID)),
            full((_HID, 3 * _HID)),
            full((1, 3 * _HID)),
            full((1, 3 * _HID)),
            full((_HID, 2 * _SEQ)),
            full((1, 2 * _SEQ)),
            full((_HID, 1)),
            full((1, 1)),
            full((_R, _HID)),
        ],
        out_specs=(
            pl.BlockSpec((_R, 2 * _SEQ), lambda i: (0, 0)),
            pl.BlockSpec((_R, 1), lambda i: (0, 0)),
        ),
        out_shape=(
            jax.ShapeDtypeStruct((_R, 2 * _SEQ), _F32),
            jax.ShapeDtypeStruct((_R, 1), _F32),
        ),
        scratch_shapes=[
            pltpu.VMEM((_R, _HID), _F32),
            pltpu.VMEM((_R, _HID), _F32),
        ],
        interpret=interpret,
    )(lhalf_all, ohs_all, w2, e36, s36, wih, whh, bih, bhh,
      wdy, bdy, wsc, bsc, hx0)


def kernel(hx, current_location, y_path, image_data, W_cnn, b_cnn, W_vel,
           b_vel, W_scf, b_scf, W_ih, W_hh, b_ih, b_hh, W_dy, b_dy,
           W_score, b_score):
    f32 = _F32

    xpad = jnp.pad(image_data, ((0, 0), (0, 0), (1, 1), (1, 1)))
    taps = [xpad[:, :, dy:dy + 159:2, dx:dx + 159:2]
            for dy in range(3) for dx in range(3)]
    x9 = jnp.stack(taps, axis=-1)
    x9 = x9.transpose(0, 2, 3, 4, 1).reshape(_NPIX, 36)
    w9 = W_cnn.transpose(2, 3, 1, 0).reshape(36, 32)

    pxa = y_path[..., 0].reshape(-1, _NA)
    pya = y_path[..., 1].reshape(-1, _NA)
    loc0 = jnp.broadcast_to(current_location[None, :, :], (_K, _BN, 2))
    prev = jnp.concatenate([loc0[:, None, :, :], y_path[:, :-1]], axis=1)

    wv0 = W_vel[:, 0].reshape(16, 1)
    wv1 = W_vel[:, 1].reshape(16, 1)
    bv = b_vel.reshape(16, 1)

    ftab = _conv_feature_table(x9, w9, b_cnn.reshape(1, 32))
    oh1, oh2, oh3, yfv, pix = _geometry(
        pxa.T, pya.T,
        y_path[..., 0].reshape(1, _NGAT), y_path[..., 1].reshape(1, _NGAT),
        prev[..., 0].reshape(1, _NGAT), prev[..., 1].reshape(1, _NGAT),
        wv0, wv1, bv)

    ohs_all = jnp.stack([oh1, oh2, oh3], axis=0)
    ohs_all = (ohs_all.reshape(3, _NA, _NB, _K, _SEQ, _BATCH)
               .transpose(4, 1, 3, 5, 0, 2).reshape(_SEQ, _R, 3, _NB))
    yfv_all = (yfv.T.reshape(_K, _SEQ, _BATCH, _NA, 16)
               .transpose(1, 3, 0, 2, 4).reshape(_SEQ, _R, 16))
    pix_flat = (pix.reshape(_K, _SEQ, _BATCH, _NA)
                .transpose(1, 3, 0, 2).reshape(_NGAT))

    feats = _sc_gather(ftab, pix_flat)
    lhalf_all = jnp.concatenate(
        [feats[:, :32].reshape(_SEQ, _R, 32), yfv_all], axis=2)

    w2 = W_scf.reshape(_HID, _NB, _HID).transpose(2, 1, 0).reshape(
        _HID, _NB * _HID)
    e36 = jnp.repeat(jnp.eye(_NB, dtype=f32), _HID, axis=1)
    s36 = jnp.tile(jnp.eye(_HID, dtype=f32), (_NB, 1))
    bih_eff = (b_ih + W_ih[:, _HID:2 * _HID] @ b_scf).reshape(1, 3 * _HID)

    hx0p = jnp.broadcast_to(
        hx.reshape(_BATCH, _NA, _HID).transpose(1, 0, 2)
        .reshape(_NA, 1, _BATCH, _HID), (_NA, _K, _BATCH, _HID)
    ).reshape(_R, _HID)

    dy_flat, sc_flat = _recurrence(
        lhalf_all, ohs_all, w2, e36, s36,
        W_ih.T, W_hh.T, bih_eff, b_hh.reshape(1, 3 * _HID),
        W_dy.T, b_dy.reshape(1, 2 * _SEQ), W_score.T, b_score.reshape(1, 1),
        hx0p)

    dy_r = (dy_flat.reshape(_NA, _K, _BATCH, 2 * _SEQ)
            .transpose(1, 2, 0, 3).reshape(_K, _BN, 2 * _SEQ))
    deltaY = dy_r.reshape(_K, _BN, 2, _SEQ).transpose(0, 3, 1, 2)
    score = (sc_flat.reshape(_NA, _K, _BATCH, 1)
             .transpose(1, 2, 0, 3).reshape(_K, _BN, 1))
    return (deltaY, score)
